# Initial kernel scaffold; baseline (speedup 1.0000x reference)
import jax
import jax.numpy as jnp
from jax.experimental import pallas as pl


def kernel(q, p, edge_index, M, W1, b1, W2, b2, gravity):
    raise NotImplementedError("write your pallas kernel here")


# speedup vs baseline: 65.9660x; 65.9660x over previous
import functools

import jax
import jax.numpy as jnp
from jax import lax
from jax.experimental import pallas as pl
from jax.experimental.pallas import tpu as pltpu
from jax.experimental.pallas import tpu_sc as plsc

NC = 2
NS = 16
C = 128
F32 = jnp.float32


_SC_PARAMS = pltpu.CompilerParams(use_tc_tiling_on_sc=False)


def _mesh():
    return plsc.VectorSubcoreMesh(core_axis_name="c", subcore_axis_name="s",
                                  num_cores=NC, num_subcores=NS)


def _sc_gs(n_pad, d, k):
    rps = n_pad // NS

    @functools.partial(
        pl.kernel,
        out_type=jax.ShapeDtypeStruct((NC, n_pad, d), F32),
        mesh=_mesh(),
        compiler_params=_SC_PARAMS,
        scratch_types=[
            pltpu.VMEM_SHARED((n_pad, d), F32),
            pltpu.VMEM_SHARED((n_pad, d), F32),
            pltpu.VMEM((k, C), jnp.int32),
            pltpu.VMEM((k, C), jnp.int32),
            pltpu.VMEM((C, d), F32),
        ],
    )
    def kern(idxg_hbm, idxs_hbm, table_hbm, zeros_hbm, out_hbm,
             table_s, acc_s, idxg_v, idxs_v, rows_v):
        c = lax.axis_index("c")
        s = lax.axis_index("s")
        wid = c * NS + s
        r0 = s * rps
        pltpu.sync_copy(table_hbm.at[pl.ds(r0, rps)], table_s.at[pl.ds(r0, rps)])
        pltpu.sync_copy(zeros_hbm.at[pl.ds(r0, rps)], acc_s.at[pl.ds(r0, rps)])
        pltpu.sync_copy(idxg_hbm.at[pl.ds(wid * k, k)], idxg_v)
        pltpu.sync_copy(idxs_hbm.at[pl.ds(wid * k, k)], idxs_v)
        plsc.subcore_barrier()

        def body(j, carry):
            pltpu.sync_copy(table_s.at[idxg_v.at[j]], rows_v)
            pltpu.sync_copy(rows_v, acc_s.at[idxs_v.at[j]], add=True)
            return carry

        lax.fori_loop(0, k, body, 0)
        plsc.subcore_barrier()
        pltpu.sync_copy(acc_s.at[pl.ds(r0, rps)], out_hbm.at[c, pl.ds(r0, rps)])

    return kern


def _sc_prep(n_pad, e_pad, d, k):
    d2 = 2 * d
    rps = n_pad // NS

    @functools.partial(
        pl.kernel,
        out_type=(jax.ShapeDtypeStruct((e_pad, d), F32),
                  jax.ShapeDtypeStruct((NC, n_pad, d2), F32),
                  jax.ShapeDtypeStruct((NC, n_pad, d2), F32)),
        mesh=_mesh(),
        compiler_params=_SC_PARAMS,
        scratch_types=[
            pltpu.VMEM_SHARED((n_pad, d), F32),
            pltpu.VMEM_SHARED((n_pad, d2), F32),
            pltpu.VMEM_SHARED((n_pad, d2), F32),
            pltpu.VMEM_SHARED((n_pad, d2), F32),
            pltpu.VMEM((k, C), jnp.int32),
            pltpu.VMEM((k, C), jnp.int32),
            pltpu.VMEM((C, d), F32),
            pltpu.VMEM((C, d), F32),
            pltpu.VMEM((C, d2), F32),
            pltpu.VMEM((C, d), F32),
        ],
    )
    def kern(src_hbm, dst_hbm, m_hbm, t2_hbm, z32_hbm, prod_hbm, outA_hbm, outB_hbm,
             m_s, t2_s, accA, accB, src_v, dst_v, ga, gb, gc, prod_v):
        c = lax.axis_index("c")
        s = lax.axis_index("s")
        wid = c * NS + s
        r0 = s * rps
        pltpu.sync_copy(m_hbm.at[pl.ds(r0, rps)], m_s.at[pl.ds(r0, rps)])
        pltpu.sync_copy(t2_hbm.at[pl.ds(r0, rps)], t2_s.at[pl.ds(r0, rps)])
        pltpu.sync_copy(z32_hbm.at[pl.ds(r0, rps)], accA.at[pl.ds(r0, rps)])
        pltpu.sync_copy(z32_hbm.at[pl.ds(r0, rps)], accB.at[pl.ds(r0, rps)])
        pltpu.sync_copy(src_hbm.at[pl.ds(wid * k, k)], src_v)
        pltpu.sync_copy(dst_hbm.at[pl.ds(wid * k, k)], dst_v)
        plsc.subcore_barrier()

        def body(j, carry):
            pltpu.sync_copy(m_s.at[src_v.at[j]], ga)
            pltpu.sync_copy(m_s.at[dst_v.at[j]], gb)
            pltpu.sync_copy(t2_s.at[src_v.at[j]], gc)
            for i in range(C):
                prod_v[i, :] = ga[i, :] * gb[i, :]
            base = (wid * k + j) * C
            pltpu.sync_copy(prod_v, prod_hbm.at[pl.ds(base, C)])
            pltpu.sync_copy(gc, accA.at[dst_v.at[j]], add=True)
            pltpu.sync_copy(gc, accB.at[src_v.at[j]], add=True)
            return carry

        lax.fori_loop(0, k, body, 0)
        plsc.subcore_barrier()
        pltpu.sync_copy(accA.at[pl.ds(r0, rps)], outA_hbm.at[c, pl.ds(r0, rps)])
        pltpu.sync_copy(accB.at[pl.ds(r0, rps)], outB_hbm.at[c, pl.ds(r0, rps)])

    return kern


def _sc_diff(n_pad, e_pad, d, k):
    rps = n_pad // NS

    @functools.partial(
        pl.kernel,
        out_type=jax.ShapeDtypeStruct((e_pad, d), F32),
        mesh=_mesh(),
        compiler_params=_SC_PARAMS,
        scratch_types=[
            pltpu.VMEM_SHARED((n_pad, d), F32),
            pltpu.VMEM((k, C), jnp.int32),
            pltpu.VMEM((k, C), jnp.int32),
            pltpu.VMEM((C, d), F32),
            pltpu.VMEM((C, d), F32),
            pltpu.VMEM((C, d), F32),
        ],
    )
    def kern(src_hbm, dst_hbm, h_hbm, out_hbm, h_s, src_v, dst_v, ga, gb, diff_v):
        c = lax.axis_index("c")
        s = lax.axis_index("s")
        wid = c * NS + s
        r0 = s * rps
        pltpu.sync_copy(h_hbm.at[pl.ds(r0, rps)], h_s.at[pl.ds(r0, rps)])
        pltpu.sync_copy(src_hbm.at[pl.ds(wid * k, k)], src_v)
        pltpu.sync_copy(dst_hbm.at[pl.ds(wid * k, k)], dst_v)
        plsc.subcore_barrier()

        def body(j, carry):
            pltpu.sync_copy(h_s.at[src_v.at[j]], ga)
            pltpu.sync_copy(h_s.at[dst_v.at[j]], gb)
            for i in range(C):
                diff_v[i, :] = ga[i, :] - gb[i, :]
            base = (wid * k + j) * C
            pltpu.sync_copy(diff_v, out_hbm.at[pl.ds(base, C)])
            return carry

        lax.fori_loop(0, k, body, 0)

    return kern


def _sc_scatter2(n_pad, e_pad, d, k):
    rps = n_pad // NS

    @functools.partial(
        pl.kernel,
        out_type=(jax.ShapeDtypeStruct((NC, n_pad, d), F32),
                  jax.ShapeDtypeStruct((NC, n_pad, d), F32)),
        mesh=_mesh(),
        compiler_params=_SC_PARAMS,
        scratch_types=[
            pltpu.VMEM_SHARED((n_pad, d), F32),
            pltpu.VMEM_SHARED((n_pad, d), F32),
            pltpu.VMEM((k, C), jnp.int32),
            pltpu.VMEM((k, C), jnp.int32),
            pltpu.VMEM((C, d), F32),
        ],
    )
    def kern(src_hbm, dst_hbm, scaled_hbm, zeros_hbm, outP_hbm, outN_hbm,
             accP, accN, src_v, dst_v, rows_v):
        c = lax.axis_index("c")
        s = lax.axis_index("s")
        wid = c * NS + s
        r0 = s * rps
        pltpu.sync_copy(zeros_hbm.at[pl.ds(r0, rps)], accP.at[pl.ds(r0, rps)])
        pltpu.sync_copy(zeros_hbm.at[pl.ds(r0, rps)], accN.at[pl.ds(r0, rps)])
        pltpu.sync_copy(src_hbm.at[pl.ds(wid * k, k)], src_v)
        pltpu.sync_copy(dst_hbm.at[pl.ds(wid * k, k)], dst_v)
        plsc.subcore_barrier()

        def body(j, carry):
            base = (wid * k + j) * C
            pltpu.sync_copy(scaled_hbm.at[pl.ds(base, C)], rows_v)
            pltpu.sync_copy(rows_v, accP.at[dst_v.at[j]], add=True)
            pltpu.sync_copy(rows_v, accN.at[src_v.at[j]], add=True)
            return carry

        lax.fori_loop(0, k, body, 0)
        plsc.subcore_barrier()
        pltpu.sync_copy(accP.at[pl.ds(r0, rps)], outP_hbm.at[c, pl.ds(r0, rps)])
        pltpu.sync_copy(accN.at[pl.ds(r0, rps)], outN_hbm.at[c, pl.ds(r0, rps)])

    return kern


def _tc_call(body, out_shapes, *args):
    return pl.pallas_call(
        body,
        out_shape=tuple(jax.ShapeDtypeStruct(s, F32) for s in out_shapes),
    )(*args)


def _tc_pre(p_pad, m_pad):
    def body(p_ref, m_ref, t2_ref):
        m = m_ref[...]
        valid = m > 0
        t = jnp.where(valid, p_ref[...] / jnp.where(valid, m, 1.0), 0.0)
        ones = jnp.where(valid, 1.0, 0.0)
        t2_ref[...] = jnp.concatenate([t, ones], axis=1)

    (t2,) = _tc_call(body, [(p_pad.shape[0], 2 * p_pad.shape[1])], p_pad, m_pad)
    return t2


def _tc1(accA, accB, q_pad, t2):
    n_pad, d = q_pad.shape

    def body(a_ref, b_ref, q_ref, t2_ref, rsin_ref, rsout_ref, x1_ref, dhdp_ref):
        st = a_ref[0, :, :d] + a_ref[1, :, :d]
        deg_in = a_ref[0, :, d:] + a_ref[1, :, d:]
        deg_out = b_ref[0, :, d:] + b_ref[1, :, d:]
        rsin_ref[...] = lax.rsqrt(jnp.maximum(deg_in, 1.0))
        rsout_ref[...] = lax.rsqrt(jnp.maximum(deg_out, 1.0))
        x1_ref[...] = q_ref[...] * rsout_ref[...]
        dhdp_ref[...] = st + t2_ref[:, :d]

    return _tc_call(body, [(n_pad, d)] * 4, accA, accB, q_pad, t2)


def _tc2(acc_sx1, rs_in, rs_out, W1, b1, W2):
    n_pad, d = rs_in.shape
    hid = W1.shape[1]

    def body(acc_ref, rsin_ref, rsout_ref, w1_ref, b1_ref, w2_ref, h2_ref, a_ref):
        sx1 = acc_ref[0] + acc_ref[1]
        a = jnp.dot(sx1 * rsin_ref[...], w1_ref[...],
                    preferred_element_type=F32) + b1_ref[...]
        a_ref[...] = a
        x2 = jnp.maximum(a, 0.0) * rsout_ref[:, :1]
        h2_ref[...] = jnp.dot(x2, w2_ref[...], preferred_element_type=F32)

    return _tc_call(body, [(n_pad, d), (n_pad, hid)],
                    acc_sx1, rs_in, rs_out, W1, b1.reshape(1, -1), W2)


def _tc3(acc_sh2, rs_in, q_pad, b2):
    n_pad, d = q_pad.shape

    def body(acc_ref, rsin_ref, q_ref, b2_ref, h_ref):
        h_ref[...] = (acc_ref[0] + acc_ref[1]) * rsin_ref[...] + b2_ref[...] + q_ref[...]

    (h,) = _tc_call(body, [(n_pad, d)], acc_sh2, rs_in, q_pad, b2.reshape(1, -1))
    return h


def _tc4(diff, prod, gravity):
    e_pad, d = diff.shape
    eb = 8192

    def body(diff_ref, prod_ref, g_ref, out_ref):
        df = diff_ref[...]
        d2 = jnp.sum(df * df, axis=1, keepdims=True)
        cc = 0.5 * g_ref[0, 0] * jnp.sum(prod_ref[...], axis=1, keepdims=True)
        r = lax.rsqrt(jnp.where(d2 > 0, d2, 1.0))
        coef = jnp.where(d2 > 0, cc * r * r * r, 0.0)
        out_ref[...] = coef * df

    return pl.pallas_call(
        body,
        grid=(e_pad // eb,),
        in_specs=[pl.BlockSpec((eb, d), lambda i: (i, 0)),
                  pl.BlockSpec((eb, d), lambda i: (i, 0)),
                  pl.BlockSpec(memory_space=pltpu.SMEM)],
        out_specs=pl.BlockSpec((eb, d), lambda i: (i, 0)),
        out_shape=jax.ShapeDtypeStruct((e_pad, d), F32),
    )(diff, prod, gravity)


def _tc5(accP, accN, rs_in):
    n_pad, d = rs_in.shape

    def body(p_ref, n_ref, rsin_ref, g_ref, y_ref):
        g = p_ref[0] + p_ref[1] - n_ref[0] - n_ref[1]
        g_ref[...] = g
        y_ref[...] = g * rsin_ref[...]

    return _tc_call(body, [(n_pad, d)] * 2, accP, accN, rs_in)


def _tc6(acc_z, a, rs_in, rs_out, W2T, W1T):
    n_pad, d = rs_in.shape

    def body(z_ref, a_ref, rsin_ref, rsout_ref, w2t_ref, w1t_ref, y2_ref):
        z = (z_ref[0] + z_ref[1]) * rsout_ref[...]
        u = jnp.dot(z, w2t_ref[...], preferred_element_type=F32)
        v = jnp.where(a_ref[...] > 0, u, 0.0)
        v16 = jnp.dot(v, w1t_ref[...], preferred_element_type=F32)
        y2_ref[...] = v16 * rsin_ref[...]

    (y2,) = _tc_call(body, [(n_pad, d)], acc_z, a, rs_in, rs_out, W2T, W1T)
    return y2


def _tc7(acc_z2, rs_out, g, dhdp, n):
    n_pad, d = rs_out.shape

    def body(z2_ref, rsout_ref, g_ref, dhdp_ref, out_ref):
        dhdq = (z2_ref[0] + z2_ref[1]) * rsout_ref[...] + g_ref[...]
        out_ref[...] = jnp.concatenate([dhdq[:n], dhdp_ref[:n]], axis=1)

    (out,) = _tc_call(body, [(n, 2 * d)], acc_z2, rs_out, g, dhdp)
    return out


def kernel(q, p, edge_index, M, W1, b1, W2, b2, gravity):
    n, d = q.shape
    e = edge_index.shape[1]
    nw = NC * NS
    n_pad = -(-(n + 1) // (NS * 8)) * NS * 8
    k = -(-e // (nw * C))
    e_pad = nw * C * k

    src = edge_index[0].astype(jnp.int32)
    dst = edge_index[1].astype(jnp.int32)
    pad = jnp.full((e_pad - e,), n, jnp.int32)
    srcp = jnp.concatenate([src, pad]).reshape(e_pad // C, C)
    dstp = jnp.concatenate([dst, pad]).reshape(e_pad // C, C)

    m = jnp.diagonal(M, axis1=1, axis2=2)
    q_pad = jnp.pad(q, ((0, n_pad - n), (0, 0)))
    p_pad = jnp.pad(p, ((0, n_pad - n), (0, 0)))
    m_pad = jnp.pad(m, ((0, n_pad - n), (0, 0)))
    zeros16 = jnp.zeros((n_pad, d), F32)
    zeros32 = jnp.zeros((n_pad, 2 * d), F32)

    sc_gs = _sc_gs(n_pad, d, k)

    t2 = _tc_pre(p_pad, m_pad)
    prod, accA, accB = _sc_prep(n_pad, e_pad, d, k)(srcp, dstp, m_pad, t2, zeros32)
    rs_in, rs_out, x1, dhdp = _tc1(accA, accB, q_pad, t2)

    acc_sx1 = sc_gs(srcp, dstp, x1, zeros16)
    h2, a = _tc2(acc_sx1, rs_in, rs_out, W1, b1, W2)
    acc_sh2 = sc_gs(srcp, dstp, h2, zeros16)
    h = _tc3(acc_sh2, rs_in, q_pad, b2)

    diff = _sc_diff(n_pad, e_pad, d, k)(srcp, dstp, h)
    scaled = _tc4(diff, prod, gravity)
    accP, accN = _sc_scatter2(n_pad, e_pad, d, k)(srcp, dstp, scaled, zeros16)
    g, y = _tc5(accP, accN, rs_in)

    acc_z = sc_gs(dstp, srcp, y, zeros16)
    y2 = _tc6(acc_z, a, rs_in, rs_out, W2.T, W1.T)
    acc_z2 = sc_gs(dstp, srcp, y2, zeros16)
    return _tc7(acc_z2, rs_out, g, dhdp, n)


# speedup vs baseline: 90.6584x; 1.3743x over previous
import functools

import jax
import jax.numpy as jnp
from jax import lax
from jax.experimental import pallas as pl
from jax.experimental.pallas import tpu as pltpu
from jax.experimental.pallas import tpu_sc as plsc

NC = 2
NS = 16
C = 128
F32 = jnp.float32


_SC_PARAMS = pltpu.CompilerParams(use_tc_tiling_on_sc=False,
                                  needs_layout_passes=False)


def _mesh():
    return plsc.VectorSubcoreMesh(core_axis_name="c", subcore_axis_name="s",
                                  num_cores=NC, num_subcores=NS)


def _sc_gs(n_pad, d, k):
    rps = n_pad // NS

    @functools.partial(
        pl.kernel,
        out_type=jax.ShapeDtypeStruct((NC, n_pad, d), F32),
        mesh=_mesh(),
        compiler_params=_SC_PARAMS,
        scratch_types=[
            pltpu.VMEM_SHARED((n_pad, d), F32),
            pltpu.VMEM_SHARED((n_pad, d), F32),
            pltpu.VMEM((k, C), jnp.int32),
            pltpu.VMEM((k, C), jnp.int32),
            pltpu.VMEM((C, d), F32),
        ],
    )
    def kern(idxg_hbm, idxs_hbm, table_hbm, zeros_hbm, out_hbm,
             table_s, acc_s, idxg_v, idxs_v, rows_v):
        c = lax.axis_index("c")
        s = lax.axis_index("s")
        wid = c * NS + s
        r0 = s * rps
        pltpu.sync_copy(table_hbm.at[pl.ds(r0, rps)], table_s.at[pl.ds(r0, rps)])
        pltpu.sync_copy(zeros_hbm.at[pl.ds(r0, rps)], acc_s.at[pl.ds(r0, rps)])
        pltpu.sync_copy(idxg_hbm.at[pl.ds(wid * k, k)], idxg_v)
        pltpu.sync_copy(idxs_hbm.at[pl.ds(wid * k, k)], idxs_v)
        plsc.subcore_barrier()

        def body(j, carry):
            pltpu.sync_copy(table_s.at[idxg_v.at[j]], rows_v)
            pltpu.sync_copy(rows_v, acc_s.at[idxs_v.at[j]], add=True)
            return carry

        lax.fori_loop(0, k, body, 0)
        plsc.subcore_barrier()
        pltpu.sync_copy(acc_s.at[pl.ds(r0, rps)], out_hbm.at[c, pl.ds(r0, rps)])

    return kern


def _sc_prep(n_pad, d, k):
    d2 = 2 * d
    rps = n_pad // NS

    @functools.partial(
        pl.kernel,
        out_type=(jax.ShapeDtypeStruct((NC, n_pad, d2), F32),
                  jax.ShapeDtypeStruct((NC, n_pad, d2), F32)),
        mesh=_mesh(),
        compiler_params=_SC_PARAMS,
        scratch_types=[
            pltpu.VMEM_SHARED((n_pad, d2), F32),
            pltpu.VMEM_SHARED((n_pad, d2), F32),
            pltpu.VMEM_SHARED((n_pad, d2), F32),
            pltpu.VMEM((k, C), jnp.int32),
            pltpu.VMEM((k, C), jnp.int32),
            pltpu.VMEM((C, d2), F32),
        ],
    )
    def kern(src_hbm, dst_hbm, t2_hbm, z32_hbm, outA_hbm, outB_hbm,
             t2_s, accA, accB, src_v, dst_v, gc):
        c = lax.axis_index("c")
        s = lax.axis_index("s")
        wid = c * NS + s
        r0 = s * rps
        pltpu.sync_copy(t2_hbm.at[pl.ds(r0, rps)], t2_s.at[pl.ds(r0, rps)])
        pltpu.sync_copy(z32_hbm.at[pl.ds(r0, rps)], accA.at[pl.ds(r0, rps)])
        pltpu.sync_copy(z32_hbm.at[pl.ds(r0, rps)], accB.at[pl.ds(r0, rps)])
        pltpu.sync_copy(src_hbm.at[pl.ds(wid * k, k)], src_v)
        pltpu.sync_copy(dst_hbm.at[pl.ds(wid * k, k)], dst_v)
        plsc.subcore_barrier()

        def body(j, carry):
            pltpu.sync_copy(t2_s.at[src_v.at[j]], gc)
            pltpu.sync_copy(gc, accA.at[dst_v.at[j]], add=True)
            pltpu.sync_copy(gc, accB.at[src_v.at[j]], add=True)
            return carry

        lax.fori_loop(0, k, body, 0)
        plsc.subcore_barrier()
        pltpu.sync_copy(accA.at[pl.ds(r0, rps)], outA_hbm.at[c, pl.ds(r0, rps)])
        pltpu.sync_copy(accB.at[pl.ds(r0, rps)], outB_hbm.at[c, pl.ds(r0, rps)])

    return kern


def _sc_edge(n_pad, d, k):
    rps = n_pad // NS

    @functools.partial(
        pl.kernel,
        out_type=jax.ShapeDtypeStruct((NC, n_pad, d), F32),
        mesh=_mesh(),
        compiler_params=_SC_PARAMS,
        scratch_types=[
            pltpu.VMEM_SHARED((n_pad, d), F32),
            pltpu.VMEM_SHARED((n_pad, d), F32),
            pltpu.VMEM_SHARED((n_pad, d), F32),
            pltpu.VMEM((k, C), jnp.int32),
            pltpu.VMEM((k, C), jnp.int32),
            pltpu.VMEM((C, d), F32),
            pltpu.VMEM((C, d), F32),
            pltpu.VMEM((C, d), F32),
            pltpu.VMEM((C, d), F32),
            pltpu.VMEM((C, d), F32),
            pltpu.VMEM((C, d), F32),
        ],
    )
    def kern(src_hbm, dst_hbm, h_hbm, m_hbm, zeros_hbm, out_hbm,
             h_s, m_s, acc_s, src_v, dst_v, ha, hb, ma, mb, pos_v, neg_v):
        c = lax.axis_index("c")
        s = lax.axis_index("s")
        wid = c * NS + s
        r0 = s * rps
        pltpu.sync_copy(h_hbm.at[pl.ds(r0, rps)], h_s.at[pl.ds(r0, rps)])
        pltpu.sync_copy(m_hbm.at[pl.ds(r0, rps)], m_s.at[pl.ds(r0, rps)])
        pltpu.sync_copy(zeros_hbm.at[pl.ds(r0, rps)], acc_s.at[pl.ds(r0, rps)])
        pltpu.sync_copy(src_hbm.at[pl.ds(wid * k, k)], src_v)
        pltpu.sync_copy(dst_hbm.at[pl.ds(wid * k, k)], dst_v)
        plsc.subcore_barrier()

        def body(j, carry):
            pltpu.sync_copy(h_s.at[src_v.at[j]], ha)
            pltpu.sync_copy(h_s.at[dst_v.at[j]], hb)
            pltpu.sync_copy(m_s.at[src_v.at[j]], ma)
            pltpu.sync_copy(m_s.at[dst_v.at[j]], mb)
            for i in range(C):
                diff = ha[i, :] - hb[i, :]
                d2 = jnp.sum(diff * diff)
                cc = jnp.sum(ma[i, :] * mb[i, :])
                yi = lax.bitcast_convert_type(
                    jnp.int32(0x5F3759DF)
                    - lax.shift_right_logical(
                        lax.bitcast_convert_type(d2, jnp.int32), 1),
                    F32)
                h2 = 0.5 * d2
                yi = yi * (1.5 - h2 * yi * yi)
                yi = yi * (1.5 - h2 * yi * yi)
                yi = yi * (1.5 - h2 * yi * yi)
                coef = cc * yi * yi * yi
                pos_v[i, :] = coef * diff
                neg_v[i, :] = (-coef) * diff
            pltpu.sync_copy(pos_v, acc_s.at[dst_v.at[j]], add=True)
            pltpu.sync_copy(neg_v, acc_s.at[src_v.at[j]], add=True)
            return carry

        lax.fori_loop(0, k, body, 0)
        plsc.subcore_barrier()
        pltpu.sync_copy(acc_s.at[pl.ds(r0, rps)], out_hbm.at[c, pl.ds(r0, rps)])

    return kern


def _tc_call(body, out_shapes, *args):
    return pl.pallas_call(
        body,
        out_shape=tuple(jax.ShapeDtypeStruct(s, F32) for s in out_shapes),
    )(*args)


def _tc_pre(p_pad, m_pad):
    def body(p_ref, m_ref, t2_ref):
        m = m_ref[...]
        valid = m > 0
        t = jnp.where(valid, p_ref[...] / jnp.where(valid, m, 1.0), 0.0)
        ones = jnp.where(valid, 1.0, 0.0)
        t2_ref[...] = jnp.concatenate([t, ones], axis=1)

    (t2,) = _tc_call(body, [(p_pad.shape[0], 2 * p_pad.shape[1])], p_pad, m_pad)
    return t2


def _tc1(accA, accB, q_pad, t2):
    n_pad, d = q_pad.shape

    def body(a_ref, b_ref, q_ref, t2_ref, rsin_ref, rsout_ref, x1_ref, dhdp_ref):
        st = a_ref[0, :, :d] + a_ref[1, :, :d]
        deg_in = a_ref[0, :, d:] + a_ref[1, :, d:]
        deg_out = b_ref[0, :, d:] + b_ref[1, :, d:]
        rsin_ref[...] = lax.rsqrt(jnp.maximum(deg_in, 1.0))
        rsout_ref[...] = lax.rsqrt(jnp.maximum(deg_out, 1.0))
        x1_ref[...] = q_ref[...] * rsout_ref[...]
        dhdp_ref[...] = st + t2_ref[:, :d]

    return _tc_call(body, [(n_pad, d)] * 4, accA, accB, q_pad, t2)


def _tc2(acc_sx1, rs_in, rs_out, W1, b1, W2):
    n_pad, d = rs_in.shape
    hid = W1.shape[1]

    def body(acc_ref, rsin_ref, rsout_ref, w1_ref, b1_ref, w2_ref, h2_ref, a_ref):
        sx1 = acc_ref[0] + acc_ref[1]
        a = jnp.dot(sx1 * rsin_ref[...], w1_ref[...],
                    preferred_element_type=F32) + b1_ref[...]
        a_ref[...] = a
        x2 = jnp.maximum(a, 0.0) * rsout_ref[:, :1]
        h2_ref[...] = jnp.dot(x2, w2_ref[...], preferred_element_type=F32)

    return _tc_call(body, [(n_pad, d), (n_pad, hid)],
                    acc_sx1, rs_in, rs_out, W1, b1.reshape(1, -1), W2)


def _tc3(acc_sh2, rs_in, q_pad, b2):
    n_pad, d = q_pad.shape

    def body(acc_ref, rsin_ref, q_ref, b2_ref, h_ref):
        h_ref[...] = (acc_ref[0] + acc_ref[1]) * rsin_ref[...] + b2_ref[...] + q_ref[...]

    (h,) = _tc_call(body, [(n_pad, d)], acc_sh2, rs_in, q_pad, b2.reshape(1, -1))
    return h


def _tc5(accG, rs_in, gravity):
    n_pad, d = rs_in.shape

    def body(acc_ref, rsin_ref, grav_ref, g_ref, y_ref):
        g = 0.5 * grav_ref[0, 0] * (acc_ref[0] + acc_ref[1])
        g_ref[...] = g
        y_ref[...] = g * rsin_ref[...]

    return _tc_call(body, [(n_pad, d)] * 2, accG, rs_in, gravity)


def _tc6(acc_z, a, rs_in, rs_out, W2T, W1T):
    n_pad, d = rs_in.shape

    def body(z_ref, a_ref, rsin_ref, rsout_ref, w2t_ref, w1t_ref, y2_ref):
        z = (z_ref[0] + z_ref[1]) * rsout_ref[...]
        u = jnp.dot(z, w2t_ref[...], preferred_element_type=F32)
        v = jnp.where(a_ref[...] > 0, u, 0.0)
        v16 = jnp.dot(v, w1t_ref[...], preferred_element_type=F32)
        y2_ref[...] = v16 * rsin_ref[...]

    (y2,) = _tc_call(body, [(n_pad, d)], acc_z, a, rs_in, rs_out, W2T, W1T)
    return y2


def _tc7(acc_z2, rs_out, g, dhdp, n):
    n_pad, d = rs_out.shape

    def body(z2_ref, rsout_ref, g_ref, dhdp_ref, out_ref):
        dhdq = (z2_ref[0] + z2_ref[1]) * rsout_ref[...] + g_ref[...]
        out_ref[...] = jnp.concatenate([dhdq[:n], dhdp_ref[:n]], axis=1)

    (out,) = _tc_call(body, [(n, 2 * d)], acc_z2, rs_out, g, dhdp)
    return out


def kernel(q, p, edge_index, M, W1, b1, W2, b2, gravity):
    n, d = q.shape
    e = edge_index.shape[1]
    nw = NC * NS
    n_pad = -(-(n + 1) // (NS * 8)) * NS * 8
    k = -(-e // (nw * C))
    e_pad = nw * C * k

    src = edge_index[0].astype(jnp.int32)
    dst = edge_index[1].astype(jnp.int32)
    pad = jnp.full((e_pad - e,), n, jnp.int32)
    srcp = jnp.concatenate([src, pad]).reshape(e_pad // C, C)
    dstp = jnp.concatenate([dst, pad]).reshape(e_pad // C, C)

    m = jnp.diagonal(M, axis1=1, axis2=2)
    q_pad = jnp.pad(q, ((0, n_pad - n), (0, 0)))
    p_pad = jnp.pad(p, ((0, n_pad - n), (0, 0)))
    m_pad = jnp.pad(m, ((0, n_pad - n), (0, 0)))
    zeros16 = jnp.zeros((n_pad, d), F32)
    zeros32 = jnp.zeros((n_pad, 2 * d), F32)

    sc_gs = _sc_gs(n_pad, d, k)

    t2 = _tc_pre(p_pad, m_pad)
    accA, accB = _sc_prep(n_pad, d, k)(srcp, dstp, t2, zeros32)
    rs_in, rs_out, x1, dhdp = _tc1(accA, accB, q_pad, t2)

    acc_sx1 = sc_gs(srcp, dstp, x1, zeros16)
    h2, a = _tc2(acc_sx1, rs_in, rs_out, W1, b1, W2)
    acc_sh2 = sc_gs(srcp, dstp, h2, zeros16)
    h = _tc3(acc_sh2, rs_in, q_pad, b2)

    accG = _sc_edge(n_pad, d, k)(srcp, dstp, h, m_pad, zeros16)
    g, y = _tc5(accG, rs_in, gravity)

    acc_z = sc_gs(dstp, srcp, y, zeros16)
    y2 = _tc6(acc_z, a, rs_in, rs_out, W2.T, W1.T)
    acc_z2 = sc_gs(dstp, srcp, y2, zeros16)
    return _tc7(acc_z2, rs_out, g, dhdp, n)


# speedup vs baseline: 120.4362x; 1.3285x over previous
import functools

import jax
import jax.numpy as jnp
from jax import lax
from jax.experimental import pallas as pl
from jax.experimental.pallas import tpu as pltpu
from jax.experimental.pallas import tpu_sc as plsc

NC = 2
NS = 16
L = 16
C = 128
F32 = jnp.float32

_SC_PARAMS = pltpu.CompilerParams(use_tc_tiling_on_sc=False,
                                  needs_layout_passes=False)


def _mesh():
    return plsc.VectorSubcoreMesh(core_axis_name="c", subcore_axis_name="s",
                                  num_cores=NC, num_subcores=NS)


def _sc_gs(n_pad, d, k):
    rps = n_pad // NS

    @functools.partial(
        pl.kernel,
        out_type=jax.ShapeDtypeStruct((NC, n_pad, d), F32),
        mesh=_mesh(),
        compiler_params=_SC_PARAMS,
        scratch_types=[
            pltpu.VMEM_SHARED((n_pad, d), F32),
            pltpu.VMEM_SHARED((n_pad, d), F32),
            pltpu.VMEM((k, C), jnp.int32),
            pltpu.VMEM((k, C), jnp.int32),
            pltpu.VMEM((C, d), F32),
        ],
    )
    def kern(idxg_hbm, idxs_hbm, table_hbm, zeros_hbm, out_hbm,
             table_s, acc_s, idxg_v, idxs_v, rows_v):
        c = lax.axis_index("c")
        s = lax.axis_index("s")
        wid = c * NS + s
        r0 = s * rps
        pltpu.sync_copy(table_hbm.at[pl.ds(r0, rps)], table_s.at[pl.ds(r0, rps)])
        pltpu.sync_copy(zeros_hbm.at[pl.ds(r0, rps)], acc_s.at[pl.ds(r0, rps)])
        pltpu.sync_copy(idxg_hbm.at[pl.ds(wid * k, k)], idxg_v)
        pltpu.sync_copy(idxs_hbm.at[pl.ds(wid * k, k)], idxs_v)
        plsc.subcore_barrier()

        def body(j, carry):
            pltpu.sync_copy(table_s.at[idxg_v.at[j]], rows_v)
            pltpu.sync_copy(rows_v, acc_s.at[idxs_v.at[j]], add=True)
            return carry

        lax.fori_loop(0, k, body, 0)
        plsc.subcore_barrier()
        pltpu.sync_copy(acc_s.at[pl.ds(r0, rps)], out_hbm.at[c, pl.ds(r0, rps)])

    return kern


def _sc_prep(n_pad, d, k):
    rps = n_pad // NS

    @functools.partial(
        pl.kernel,
        out_type=(jax.ShapeDtypeStruct((NC, n_pad, d), F32),
                  jax.ShapeDtypeStruct((NC, n_pad, d), F32),
                  jax.ShapeDtypeStruct((NC, n_pad, d), F32)),
        mesh=_mesh(),
        compiler_params=_SC_PARAMS,
        scratch_types=[
            pltpu.VMEM_SHARED((n_pad, d), F32),
            pltpu.VMEM_SHARED((n_pad, d), F32),
            pltpu.VMEM_SHARED((n_pad, d), F32),
            pltpu.VMEM_SHARED((n_pad, d), F32),
            pltpu.VMEM((k, C), jnp.int32),
            pltpu.VMEM((k, C), jnp.int32),
            pltpu.VMEM((C, d), F32),
            pltpu.VMEM((C, d), F32),
        ],
    )
    def kern(src_hbm, dst_hbm, t_hbm, zeros_hbm, outSt_hbm, outDi_hbm, outDo_hbm,
             t_s, accSt, accDi, accDo, src_v, dst_v, gt, ones_v):
        c = lax.axis_index("c")
        s = lax.axis_index("s")
        wid = c * NS + s
        r0 = s * rps
        pltpu.sync_copy(t_hbm.at[pl.ds(r0, rps)], t_s.at[pl.ds(r0, rps)])
        pltpu.sync_copy(zeros_hbm.at[pl.ds(r0, rps)], accSt.at[pl.ds(r0, rps)])
        pltpu.sync_copy(zeros_hbm.at[pl.ds(r0, rps)], accDi.at[pl.ds(r0, rps)])
        pltpu.sync_copy(zeros_hbm.at[pl.ds(r0, rps)], accDo.at[pl.ds(r0, rps)])
        pltpu.sync_copy(src_hbm.at[pl.ds(wid * k, k)], src_v)
        pltpu.sync_copy(dst_hbm.at[pl.ds(wid * k, k)], dst_v)
        one = jnp.ones((L,), F32)
        for i in range(C):
            ones_v[i, :] = one
        plsc.subcore_barrier()

        def body(j, carry):
            pltpu.sync_copy(t_s.at[src_v.at[j]], gt)
            pltpu.sync_copy(gt, accSt.at[dst_v.at[j]], add=True)
            pltpu.sync_copy(ones_v, accDi.at[dst_v.at[j]], add=True)
            pltpu.sync_copy(ones_v, accDo.at[src_v.at[j]], add=True)
            return carry

        lax.fori_loop(0, k, body, 0)
        plsc.subcore_barrier()
        pltpu.sync_copy(accSt.at[pl.ds(r0, rps)], outSt_hbm.at[c, pl.ds(r0, rps)])
        pltpu.sync_copy(accDi.at[pl.ds(r0, rps)], outDi_hbm.at[c, pl.ds(r0, rps)])
        pltpu.sync_copy(accDo.at[pl.ds(r0, rps)], outDo_hbm.at[c, pl.ds(r0, rps)])

    return kern


def _sc_edge(n_pad, d, k):
    rps = n_pad // NS
    ngrp = C // L

    @functools.partial(
        pl.kernel,
        out_type=jax.ShapeDtypeStruct((NC, n_pad, d), F32),
        mesh=_mesh(),
        compiler_params=_SC_PARAMS,
        scratch_types=[
            pltpu.VMEM_SHARED((n_pad, d), F32),
            pltpu.VMEM_SHARED((n_pad, d), F32),
            pltpu.VMEM_SHARED((n_pad, d), F32),
            pltpu.VMEM((k, C), jnp.int32),
            pltpu.VMEM((k, C), jnp.int32),
            pltpu.VMEM((C, d), F32),
            pltpu.VMEM((C, d), F32),
            pltpu.VMEM((C, d), F32),
            pltpu.VMEM((C, d), F32),
            pltpu.VMEM((C, d), F32),
            pltpu.VMEM((C, d), F32),
        ],
    )
    def kern(src_hbm, dst_hbm, h_hbm, m_hbm, zeros_hbm, out_hbm,
             h_s, m_s, acc_s, src_v, dst_v, ha, hb, ma, mb, pos_v, neg_v):
        c = lax.axis_index("c")
        s = lax.axis_index("s")
        wid = c * NS + s
        r0 = s * rps
        pltpu.sync_copy(h_hbm.at[pl.ds(r0, rps)], h_s.at[pl.ds(r0, rps)])
        pltpu.sync_copy(m_hbm.at[pl.ds(r0, rps)], m_s.at[pl.ds(r0, rps)])
        pltpu.sync_copy(zeros_hbm.at[pl.ds(r0, rps)], acc_s.at[pl.ds(r0, rps)])
        pltpu.sync_copy(src_hbm.at[pl.ds(wid * k, k)], src_v)
        pltpu.sync_copy(dst_hbm.at[pl.ds(wid * k, k)], dst_v)
        plsc.subcore_barrier()
        iota = lax.iota(jnp.int32, L)

        def body(j, carry):
            pltpu.sync_copy(h_s.at[src_v.at[j]], ha)
            pltpu.sync_copy(h_s.at[dst_v.at[j]], hb)
            pltpu.sync_copy(m_s.at[src_v.at[j]], ma)
            pltpu.sync_copy(m_s.at[dst_v.at[j]], mb)
            for g in range(ngrp):
                rows = iota + (g * L)
                d2 = jnp.zeros((L,), F32)
                cc = jnp.zeros((L,), F32)
                diffs = []
                for kk in range(d):
                    cols = jnp.full((L,), kk, jnp.int32)
                    df = (plsc.load_gather(ha, [rows, cols])
                          - plsc.load_gather(hb, [rows, cols]))
                    diffs.append(df)
                    d2 = d2 + df * df
                    cc = cc + (plsc.load_gather(ma, [rows, cols])
                               * plsc.load_gather(mb, [rows, cols]))
                yi = lax.bitcast_convert_type(
                    jnp.full((L,), 0x5F3759DF, jnp.int32)
                    - lax.shift_right_logical(
                        lax.bitcast_convert_type(d2, jnp.int32), 1),
                    F32)
                hd2 = 0.5 * d2
                yi = yi * (1.5 - hd2 * yi * yi)
                yi = yi * (1.5 - hd2 * yi * yi)
                yi = yi * (1.5 - hd2 * yi * yi)
                coef = cc * yi * yi * yi
                for kk in range(d):
                    cols = jnp.full((L,), kk, jnp.int32)
                    v = coef * diffs[kk]
                    plsc.store_scatter(pos_v, [rows, cols], v)
                    plsc.store_scatter(neg_v, [rows, cols], -v)
            pltpu.sync_copy(pos_v, acc_s.at[dst_v.at[j]], add=True)
            pltpu.sync_copy(neg_v, acc_s.at[src_v.at[j]], add=True)
            return carry

        lax.fori_loop(0, k, body, 0)
        plsc.subcore_barrier()
        pltpu.sync_copy(acc_s.at[pl.ds(r0, rps)], out_hbm.at[c, pl.ds(r0, rps)])

    return kern


def _tc_call(body, out_shapes, *args):
    return pl.pallas_call(
        body,
        out_shape=tuple(jax.ShapeDtypeStruct(s, F32) for s in out_shapes),
    )(*args)


def _tc_pre(p128, m128):
    def body(p_ref, m_ref, t_ref):
        m = m_ref[...]
        valid = m > 0
        t_ref[...] = jnp.where(valid, p_ref[...] / jnp.where(valid, m, 1.0), 0.0)

    (t,) = _tc_call(body, [p128.shape], p128, m128)
    return t


def _tc1(accSt, accDi, accDo, q128, t128):
    def body(st_ref, di_ref, do_ref, q_ref, t_ref,
             rsin_ref, rsout_ref, x1_ref, dhdp_ref):
        rsin_ref[...] = lax.rsqrt(jnp.maximum(di_ref[0] + di_ref[1], 1.0))
        rsout_ref[...] = lax.rsqrt(jnp.maximum(do_ref[0] + do_ref[1], 1.0))
        x1_ref[...] = q_ref[...] * rsout_ref[...]
        dhdp_ref[...] = st_ref[0] + st_ref[1] + t_ref[...]

    return _tc_call(body, [q128.shape] * 4, accSt, accDi, accDo, q128, t128)


def _tc2(acc, rs_in, rs_out, W1big, b1big, W2big):
    r128, _ = rs_in.shape
    hidb = W1big.shape[1]

    def body(acc_ref, rsin_ref, rsout_ref, w1_ref, b1_ref, w2_ref,
             h2_ref, a_ref):
        xw = (acc_ref[0] + acc_ref[1]) * rsin_ref[...]
        a = jnp.dot(xw, w1_ref[...], preferred_element_type=F32) + b1_ref[...]
        a_ref[...] = a
        h2_ref[...] = jnp.dot(jnp.maximum(a, 0.0), w2_ref[...],
                              preferred_element_type=F32) * rsout_ref[...]

    return _tc_call(body, [rs_in.shape, (r128, hidb)],
                    acc, rs_in, rs_out, W1big, b1big, W2big)


def _tc3(acc, rs_in, q128, b2big):
    def body(acc_ref, rsin_ref, q_ref, b2_ref, h_ref):
        h_ref[...] = ((acc_ref[0] + acc_ref[1]) * rsin_ref[...]
                      + b2_ref[...] + q_ref[...])

    (h,) = _tc_call(body, [q128.shape], acc, rs_in, q128, b2big)
    return h


def _tc5(accG, rs_in, gravity):
    def body(acc_ref, rsin_ref, grav_ref, g_ref, y_ref):
        g = 0.5 * grav_ref[0, 0] * (acc_ref[0] + acc_ref[1])
        g_ref[...] = g
        y_ref[...] = g * rsin_ref[...]

    return _tc_call(body, [rs_in.shape] * 2, accG, rs_in, gravity)


def _tc6(acc, a_big, rs_in, rs_out, W2Tbig, W1Tbig):
    def body(acc_ref, a_ref, rsin_ref, rsout_ref, w2t_ref, w1t_ref, y2_ref):
        zw = (acc_ref[0] + acc_ref[1]) * rsout_ref[...]
        u = jnp.dot(zw, w2t_ref[...], preferred_element_type=F32)
        v = jnp.where(a_ref[...] > 0, u, 0.0)
        y2_ref[...] = jnp.dot(v, w1t_ref[...],
                              preferred_element_type=F32) * rsin_ref[...]

    (y2,) = _tc_call(body, [rs_in.shape], acc, a_big, rs_in, rs_out,
                     W2Tbig, W1Tbig)
    return y2


def _tc7(acc, rs_out, g128):
    def body(acc_ref, rsout_ref, g_ref, dhdq_ref):
        dhdq_ref[...] = (acc_ref[0] + acc_ref[1]) * rsout_ref[...] + g_ref[...]

    (dhdq,) = _tc_call(body, [rs_out.shape], acc, rs_out, g128)
    return dhdq


def kernel(q, p, edge_index, M, W1, b1, W2, b2, gravity):
    n, d = q.shape
    e = edge_index.shape[1]
    nw = NC * NS
    n_pad = -(-(n + 1) // (NS * 8)) * NS * 8
    k = -(-e // (nw * C))
    e_pad = nw * C * k
    r128 = n_pad * d // 128
    nb = 128 // d

    src = edge_index[0].astype(jnp.int32)
    dst = edge_index[1].astype(jnp.int32)
    pad = jnp.full((e_pad - e,), n, jnp.int32)
    srcp = jnp.concatenate([src, pad]).reshape(e_pad // C, C)
    dstp = jnp.concatenate([dst, pad]).reshape(e_pad // C, C)

    m = jnp.diagonal(M, axis1=1, axis2=2)
    m_pad = jnp.pad(m, ((0, n_pad - n), (0, 0)))
    q128 = jnp.pad(q, ((0, n_pad - n), (0, 0))).reshape(r128, 128)
    p128 = jnp.pad(p, ((0, n_pad - n), (0, 0))).reshape(r128, 128)
    m128 = m_pad.reshape(r128, 128)
    zeros16 = jnp.zeros((n_pad, d), F32)

    eye = jnp.eye(nb, dtype=F32)
    W1big = jnp.kron(eye, W1)
    W2big = jnp.kron(eye, W2)
    W2Tbig = jnp.kron(eye, W2.T)
    W1Tbig = jnp.kron(eye, W1.T)
    b1big = jnp.tile(b1, nb).reshape(1, nb * b1.shape[0])
    b2big = jnp.tile(b2, nb).reshape(1, 128)

    def v128(acc):
        return acc.reshape(NC, r128, 128)

    def v16(x):
        return x.reshape(n_pad, d)

    sc_gs = _sc_gs(n_pad, d, k)

    t128 = _tc_pre(p128, m128)
    accSt, accDi, accDo = _sc_prep(n_pad, d, k)(srcp, dstp, v16(t128), zeros16)
    rs_in, rs_out, x1, dhdp = _tc1(v128(accSt), v128(accDi), v128(accDo),
                                   q128, t128)

    acc1 = sc_gs(srcp, dstp, v16(x1), zeros16)
    h2, a_big = _tc2(v128(acc1), rs_in, rs_out, W1big, b1big, W2big)
    acc2 = sc_gs(srcp, dstp, v16(h2), zeros16)
    h = _tc3(v128(acc2), rs_in, q128, b2big)

    accG = _sc_edge(n_pad, d, k)(srcp, dstp, v16(h), m_pad, zeros16)
    g128, y = _tc5(v128(accG), rs_in, gravity)

    accZ = sc_gs(dstp, srcp, v16(y), zeros16)
    y2 = _tc6(v128(accZ), a_big, rs_in, rs_out, W2Tbig, W1Tbig)
    accZ2 = sc_gs(dstp, srcp, v16(y2), zeros16)
    dhdq = _tc7(v128(accZ2), rs_out, g128)

    return jnp.concatenate([v16(dhdq)[:n], v16(dhdp)[:n]], axis=1)


# speedup vs baseline: 123.7379x; 1.0274x over previous
import functools

import jax
import jax.numpy as jnp
from jax import lax
from jax.experimental import pallas as pl
from jax.experimental.pallas import tpu as pltpu
from jax.experimental.pallas import tpu_sc as plsc

NC = 2
NS = 16
L = 16
C = 128
F32 = jnp.float32

_SC_PARAMS = pltpu.CompilerParams(use_tc_tiling_on_sc=False,
                                  needs_layout_passes=False)


def _mesh():
    return plsc.VectorSubcoreMesh(core_axis_name="c", subcore_axis_name="s",
                                  num_cores=NC, num_subcores=NS)


def _sc_gs(n_pad, d, k):
    rps = n_pad // NS

    @functools.partial(
        pl.kernel,
        out_type=jax.ShapeDtypeStruct((NC, n_pad, d), F32),
        mesh=_mesh(),
        compiler_params=_SC_PARAMS,
        scratch_types=[
            pltpu.VMEM_SHARED((n_pad, d), F32),
            pltpu.VMEM_SHARED((n_pad, d), F32),
            pltpu.VMEM((k, C), jnp.int32),
            pltpu.VMEM((k, C), jnp.int32),
            pltpu.VMEM((C, d), F32),
        ],
    )
    def kern(idxg_hbm, idxs_hbm, table_hbm, zeros_hbm, out_hbm,
             table_s, acc_s, idxg_v, idxs_v, rows_v):
        c = lax.axis_index("c")
        s = lax.axis_index("s")
        wid = c * NS + s
        r0 = s * rps
        pltpu.sync_copy(table_hbm.at[pl.ds(r0, rps)], table_s.at[pl.ds(r0, rps)])
        pltpu.sync_copy(zeros_hbm.at[pl.ds(r0, rps)], acc_s.at[pl.ds(r0, rps)])
        pltpu.sync_copy(idxg_hbm.at[pl.ds(wid * k, k)], idxg_v)
        pltpu.sync_copy(idxs_hbm.at[pl.ds(wid * k, k)], idxs_v)
        plsc.subcore_barrier()

        def body(j, carry):
            pltpu.sync_copy(table_s.at[idxg_v.at[j]], rows_v)
            pltpu.sync_copy(rows_v, acc_s.at[idxs_v.at[j]], add=True)
            return carry

        lax.fori_loop(0, k, body, 0)
        plsc.subcore_barrier()
        pltpu.sync_copy(acc_s.at[pl.ds(r0, rps)], out_hbm.at[c, pl.ds(r0, rps)])

    return kern


def _sc_prep(n_pad, d, k):
    rps = n_pad // NS

    @functools.partial(
        pl.kernel,
        out_type=(jax.ShapeDtypeStruct((NC, n_pad, d), F32),
                  jax.ShapeDtypeStruct((NC, n_pad, d), F32),
                  jax.ShapeDtypeStruct((NC, n_pad, d), F32)),
        mesh=_mesh(),
        compiler_params=_SC_PARAMS,
        scratch_types=[
            pltpu.VMEM_SHARED((n_pad, d), F32),
            pltpu.VMEM_SHARED((n_pad, d), F32),
            pltpu.VMEM_SHARED((n_pad, d), F32),
            pltpu.VMEM_SHARED((n_pad, d), F32),
            pltpu.VMEM((k, C), jnp.int32),
            pltpu.VMEM((k, C), jnp.int32),
            pltpu.VMEM((C, d), F32),
            pltpu.VMEM((C, d), F32),
        ],
    )
    def kern(src_hbm, dst_hbm, t_hbm, zeros_hbm, outSt_hbm, outDi_hbm, outDo_hbm,
             t_s, accSt, accDi, accDo, src_v, dst_v, gt, ones_v):
        c = lax.axis_index("c")
        s = lax.axis_index("s")
        wid = c * NS + s
        r0 = s * rps
        pltpu.sync_copy(t_hbm.at[pl.ds(r0, rps)], t_s.at[pl.ds(r0, rps)])
        pltpu.sync_copy(zeros_hbm.at[pl.ds(r0, rps)], accSt.at[pl.ds(r0, rps)])
        pltpu.sync_copy(zeros_hbm.at[pl.ds(r0, rps)], accDi.at[pl.ds(r0, rps)])
        pltpu.sync_copy(zeros_hbm.at[pl.ds(r0, rps)], accDo.at[pl.ds(r0, rps)])
        pltpu.sync_copy(src_hbm.at[pl.ds(wid * k, k)], src_v)
        pltpu.sync_copy(dst_hbm.at[pl.ds(wid * k, k)], dst_v)
        one = jnp.ones((L,), F32)
        for i in range(C):
            ones_v[i, :] = one
        plsc.subcore_barrier()

        def body(j, carry):
            pltpu.sync_copy(t_s.at[src_v.at[j]], gt)
            pltpu.sync_copy(gt, accSt.at[dst_v.at[j]], add=True)
            pltpu.sync_copy(ones_v, accDi.at[dst_v.at[j]], add=True)
            pltpu.sync_copy(ones_v, accDo.at[src_v.at[j]], add=True)
            return carry

        lax.fori_loop(0, k, body, 0)
        plsc.subcore_barrier()
        pltpu.sync_copy(accSt.at[pl.ds(r0, rps)], outSt_hbm.at[c, pl.ds(r0, rps)])
        pltpu.sync_copy(accDi.at[pl.ds(r0, rps)], outDi_hbm.at[c, pl.ds(r0, rps)])
        pltpu.sync_copy(accDo.at[pl.ds(r0, rps)], outDo_hbm.at[c, pl.ds(r0, rps)])

    return kern


def _sc_edge(n_pad, d, k):
    rps = n_pad // NS
    ngrp = C // L

    @functools.partial(
        pl.kernel,
        out_type=jax.ShapeDtypeStruct((NC, n_pad, d), F32),
        mesh=_mesh(),
        compiler_params=_SC_PARAMS,
        scratch_types=[
            pltpu.VMEM_SHARED((n_pad, d), F32),
            pltpu.VMEM_SHARED((n_pad, d), F32),
            pltpu.VMEM_SHARED((n_pad, d), F32),
            pltpu.VMEM((k, C), jnp.int32),
            pltpu.VMEM((k, C), jnp.int32),
            pltpu.VMEM((C, d), F32),
            pltpu.VMEM((C, d), F32),
            pltpu.VMEM((C, d), F32),
            pltpu.VMEM((C, d), F32),
            pltpu.VMEM((C, d), F32),
            pltpu.VMEM((C, d), F32),
        ],
    )
    def kern(src_hbm, dst_hbm, h_hbm, m_hbm, zeros_hbm, out_hbm,
             h_s, m_s, acc_s, src_v, dst_v, ha, hb, ma, mb, pos_v, neg_v):
        c = lax.axis_index("c")
        s = lax.axis_index("s")
        wid = c * NS + s
        r0 = s * rps
        pltpu.sync_copy(h_hbm.at[pl.ds(r0, rps)], h_s.at[pl.ds(r0, rps)])
        pltpu.sync_copy(m_hbm.at[pl.ds(r0, rps)], m_s.at[pl.ds(r0, rps)])
        pltpu.sync_copy(zeros_hbm.at[pl.ds(r0, rps)], acc_s.at[pl.ds(r0, rps)])
        pltpu.sync_copy(src_hbm.at[pl.ds(wid * k, k)], src_v)
        pltpu.sync_copy(dst_hbm.at[pl.ds(wid * k, k)], dst_v)
        plsc.subcore_barrier()
        iota = lax.iota(jnp.int32, L)
        shuf = [lax.bitwise_xor(iota, jnp.int32(r)) for r in (1, 2, 4, 8)]

        dnums = lax.GatherDimensionNumbers(
            offset_dims=(), collapsed_slice_dims=(0,), start_index_map=(0,))

        def hsum(v):
            for idx in shuf:
                v = v + lax.gather(
                    v, idx[:, None], dnums, (1,),
                    mode=lax.GatherScatterMode.PROMISE_IN_BOUNDS)
            return v

        def body(j, carry):
            pltpu.sync_copy(h_s.at[src_v.at[j]], ha)
            pltpu.sync_copy(h_s.at[dst_v.at[j]], hb)
            pltpu.sync_copy(m_s.at[src_v.at[j]], ma)
            pltpu.sync_copy(m_s.at[dst_v.at[j]], mb)
            for i in range(C):
                df = ha[i, :] - hb[i, :]
                d2 = hsum(df * df)
                cc = hsum(ma[i, :] * mb[i, :])
                yi = lax.bitcast_convert_type(
                    jnp.full((L,), 0x5F3759DF, jnp.int32)
                    - lax.shift_right_logical(
                        lax.bitcast_convert_type(d2, jnp.int32), 1),
                    F32)
                hd2 = 0.5 * d2
                yi = yi * (1.5 - hd2 * yi * yi)
                yi = yi * (1.5 - hd2 * yi * yi)
                coef = cc * yi * yi * yi
                v = coef * df
                pos_v[i, :] = v
                neg_v[i, :] = -v
            pltpu.sync_copy(pos_v, acc_s.at[dst_v.at[j]], add=True)
            pltpu.sync_copy(neg_v, acc_s.at[src_v.at[j]], add=True)
            return carry

        lax.fori_loop(0, k, body, 0)
        plsc.subcore_barrier()
        pltpu.sync_copy(acc_s.at[pl.ds(r0, rps)], out_hbm.at[c, pl.ds(r0, rps)])

    return kern


def _tc_call(body, out_shapes, *args):
    return pl.pallas_call(
        body,
        out_shape=tuple(jax.ShapeDtypeStruct(s, F32) for s in out_shapes),
    )(*args)


def _tc_pre(p128, m128):
    def body(p_ref, m_ref, t_ref):
        m = m_ref[...]
        valid = m > 0
        t_ref[...] = jnp.where(valid, p_ref[...] / jnp.where(valid, m, 1.0), 0.0)

    (t,) = _tc_call(body, [p128.shape], p128, m128)
    return t


def _tc1(accSt, accDi, accDo, q128, t128):
    def body(st_ref, di_ref, do_ref, q_ref, t_ref,
             rsin_ref, rsout_ref, x1_ref, dhdp_ref):
        rsin_ref[...] = lax.rsqrt(jnp.maximum(di_ref[0] + di_ref[1], 1.0))
        rsout_ref[...] = lax.rsqrt(jnp.maximum(do_ref[0] + do_ref[1], 1.0))
        x1_ref[...] = q_ref[...] * rsout_ref[...]
        dhdp_ref[...] = st_ref[0] + st_ref[1] + t_ref[...]

    return _tc_call(body, [q128.shape] * 4, accSt, accDi, accDo, q128, t128)


def _tc2(acc, rs_in, rs_out, W1big, b1big, W2big):
    r128, _ = rs_in.shape
    hidb = W1big.shape[1]

    def body(acc_ref, rsin_ref, rsout_ref, w1_ref, b1_ref, w2_ref,
             h2_ref, a_ref):
        xw = (acc_ref[0] + acc_ref[1]) * rsin_ref[...]
        a = jnp.dot(xw, w1_ref[...], preferred_element_type=F32) + b1_ref[...]
        a_ref[...] = a
        h2_ref[...] = jnp.dot(jnp.maximum(a, 0.0), w2_ref[...],
                              preferred_element_type=F32) * rsout_ref[...]

    return _tc_call(body, [rs_in.shape, (r128, hidb)],
                    acc, rs_in, rs_out, W1big, b1big, W2big)


def _tc3(acc, rs_in, q128, b2big):
    def body(acc_ref, rsin_ref, q_ref, b2_ref, h_ref):
        h_ref[...] = ((acc_ref[0] + acc_ref[1]) * rsin_ref[...]
                      + b2_ref[...] + q_ref[...])

    (h,) = _tc_call(body, [q128.shape], acc, rs_in, q128, b2big)
    return h


def _tc5(accG, rs_in, gravity):
    def body(acc_ref, rsin_ref, grav_ref, g_ref, y_ref):
        g = 0.5 * grav_ref[0, 0] * (acc_ref[0] + acc_ref[1])
        g_ref[...] = g
        y_ref[...] = g * rsin_ref[...]

    return _tc_call(body, [rs_in.shape] * 2, accG, rs_in, gravity)


def _tc6(acc, a_big, rs_in, rs_out, W2Tbig, W1Tbig):
    def body(acc_ref, a_ref, rsin_ref, rsout_ref, w2t_ref, w1t_ref, y2_ref):
        zw = (acc_ref[0] + acc_ref[1]) * rsout_ref[...]
        u = jnp.dot(zw, w2t_ref[...], preferred_element_type=F32)
        v = jnp.where(a_ref[...] > 0, u, 0.0)
        y2_ref[...] = jnp.dot(v, w1t_ref[...],
                              preferred_element_type=F32) * rsin_ref[...]

    (y2,) = _tc_call(body, [rs_in.shape], acc, a_big, rs_in, rs_out,
                     W2Tbig, W1Tbig)
    return y2


def _tc7(acc, rs_out, g128):
    def body(acc_ref, rsout_ref, g_ref, dhdq_ref):
        dhdq_ref[...] = (acc_ref[0] + acc_ref[1]) * rsout_ref[...] + g_ref[...]

    (dhdq,) = _tc_call(body, [rs_out.shape], acc, rs_out, g128)
    return dhdq


def kernel(q, p, edge_index, M, W1, b1, W2, b2, gravity):
    n, d = q.shape
    e = edge_index.shape[1]
    nw = NC * NS
    n_pad = -(-(n + 1) // (NS * 8)) * NS * 8
    k = -(-e // (nw * C))
    e_pad = nw * C * k
    r128 = n_pad * d // 128
    nb = 128 // d

    src = edge_index[0].astype(jnp.int32)
    dst = edge_index[1].astype(jnp.int32)
    pad = jnp.full((e_pad - e,), n, jnp.int32)
    srcp = jnp.concatenate([src, pad]).reshape(e_pad // C, C)
    dstp = jnp.concatenate([dst, pad]).reshape(e_pad // C, C)

    m = jnp.diagonal(M, axis1=1, axis2=2)
    m_pad = jnp.pad(m, ((0, n_pad - n), (0, 0)))
    q128 = jnp.pad(q, ((0, n_pad - n), (0, 0))).reshape(r128, 128)
    p128 = jnp.pad(p, ((0, n_pad - n), (0, 0))).reshape(r128, 128)
    m128 = m_pad.reshape(r128, 128)
    zeros16 = jnp.zeros((n_pad, d), F32)

    eye = jnp.eye(nb, dtype=F32)
    W1big = jnp.kron(eye, W1)
    W2big = jnp.kron(eye, W2)
    W2Tbig = jnp.kron(eye, W2.T)
    W1Tbig = jnp.kron(eye, W1.T)
    b1big = jnp.tile(b1, nb).reshape(1, nb * b1.shape[0])
    b2big = jnp.tile(b2, nb).reshape(1, 128)

    def v128(acc):
        return acc.reshape(NC, r128, 128)

    def v16(x):
        return x.reshape(n_pad, d)

    sc_gs = _sc_gs(n_pad, d, k)

    t128 = _tc_pre(p128, m128)
    accSt, accDi, accDo = _sc_prep(n_pad, d, k)(srcp, dstp, v16(t128), zeros16)
    rs_in, rs_out, x1, dhdp = _tc1(v128(accSt), v128(accDi), v128(accDo),
                                   q128, t128)

    acc1 = sc_gs(srcp, dstp, v16(x1), zeros16)
    h2, a_big = _tc2(v128(acc1), rs_in, rs_out, W1big, b1big, W2big)
    acc2 = sc_gs(srcp, dstp, v16(h2), zeros16)
    h = _tc3(v128(acc2), rs_in, q128, b2big)

    accG = _sc_edge(n_pad, d, k)(srcp, dstp, v16(h), m_pad, zeros16)
    g128, y = _tc5(v128(accG), rs_in, gravity)

    accZ = sc_gs(dstp, srcp, v16(y), zeros16)
    y2 = _tc6(v128(accZ), a_big, rs_in, rs_out, W2Tbig, W1Tbig)
    accZ2 = sc_gs(dstp, srcp, v16(y2), zeros16)
    dhdq = _tc7(v128(accZ2), rs_out, g128)

    return jnp.concatenate([v16(dhdq)[:n], v16(dhdp)[:n]], axis=1)


# speedup vs baseline: 148.6036x; 1.2010x over previous
import functools

import jax
import jax.numpy as jnp
from jax import lax
from jax.experimental import pallas as pl
from jax.experimental.pallas import tpu as pltpu
from jax.experimental.pallas import tpu_sc as plsc

NC = 2
NS = 16
L = 16
C = 128
F32 = jnp.float32

_SC_PARAMS = pltpu.CompilerParams(use_tc_tiling_on_sc=False,
                                  needs_layout_passes=False)


def _mesh():
    return plsc.VectorSubcoreMesh(core_axis_name="c", subcore_axis_name="s",
                                  num_cores=NC, num_subcores=NS)


def _sc_gs(n_pad, d, k):
    rps = n_pad // NS

    @functools.partial(
        pl.kernel,
        out_type=jax.ShapeDtypeStruct((NC, n_pad, d), F32),
        mesh=_mesh(),
        compiler_params=_SC_PARAMS,
        scratch_types=[
            pltpu.VMEM_SHARED((n_pad, d), F32),
            pltpu.VMEM_SHARED((n_pad, d), F32),
            pltpu.VMEM((k, C), jnp.int32),
            pltpu.VMEM((k, C), jnp.int32),
            pltpu.VMEM((C, d), F32),
        ],
    )
    def kern(idxg_hbm, idxs_hbm, table_hbm, zeros_hbm, out_hbm,
             table_s, acc_s, idxg_v, idxs_v, rows_v):
        c = lax.axis_index("c")
        s = lax.axis_index("s")
        wid = c * NS + s
        r0 = s * rps
        pltpu.sync_copy(table_hbm.at[pl.ds(r0, rps)], table_s.at[pl.ds(r0, rps)])
        pltpu.sync_copy(zeros_hbm.at[pl.ds(r0, rps)], acc_s.at[pl.ds(r0, rps)])
        pltpu.sync_copy(idxg_hbm.at[pl.ds(wid * k, k)], idxg_v)
        pltpu.sync_copy(idxs_hbm.at[pl.ds(wid * k, k)], idxs_v)
        plsc.subcore_barrier()

        def body(j, carry):
            pltpu.sync_copy(table_s.at[idxg_v.at[j]], rows_v)
            pltpu.sync_copy(rows_v, acc_s.at[idxs_v.at[j]], add=True)
            return carry

        lax.fori_loop(0, k, body, 0)
        plsc.subcore_barrier()
        pltpu.sync_copy(acc_s.at[pl.ds(r0, rps)], out_hbm.at[c, pl.ds(r0, rps)])

    return kern


def _sc_prep(n_pad, d, k):
    rps = n_pad // NS

    @functools.partial(
        pl.kernel,
        out_type=(jax.ShapeDtypeStruct((NC, n_pad, d), F32),
                  jax.ShapeDtypeStruct((NC, n_pad, d), F32),
                  jax.ShapeDtypeStruct((NC, n_pad, d), F32)),
        mesh=_mesh(),
        compiler_params=_SC_PARAMS,
        scratch_types=[
            pltpu.VMEM_SHARED((n_pad, d), F32),
            pltpu.VMEM_SHARED((n_pad, d), F32),
            pltpu.VMEM_SHARED((n_pad, d), F32),
            pltpu.VMEM_SHARED((n_pad, d), F32),
            pltpu.VMEM((k, C), jnp.int32),
            pltpu.VMEM((k, C), jnp.int32),
            pltpu.VMEM((C, d), F32),
            pltpu.VMEM((C, d), F32),
        ],
    )
    def kern(src_hbm, dst_hbm, t_hbm, zeros_hbm, outSt_hbm, outDi_hbm, outDo_hbm,
             t_s, accSt, accDi, accDo, src_v, dst_v, gt, ones_v):
        c = lax.axis_index("c")
        s = lax.axis_index("s")
        wid = c * NS + s
        r0 = s * rps
        pltpu.sync_copy(t_hbm.at[pl.ds(r0, rps)], t_s.at[pl.ds(r0, rps)])
        pltpu.sync_copy(zeros_hbm.at[pl.ds(r0, rps)], accSt.at[pl.ds(r0, rps)])
        pltpu.sync_copy(zeros_hbm.at[pl.ds(r0, rps)], accDi.at[pl.ds(r0, rps)])
        pltpu.sync_copy(zeros_hbm.at[pl.ds(r0, rps)], accDo.at[pl.ds(r0, rps)])
        pltpu.sync_copy(src_hbm.at[pl.ds(wid * k, k)], src_v)
        pltpu.sync_copy(dst_hbm.at[pl.ds(wid * k, k)], dst_v)
        one = jnp.ones((L,), F32)
        for i in range(C):
            ones_v[i, :] = one
        plsc.subcore_barrier()

        def body(j, carry):
            pltpu.sync_copy(t_s.at[src_v.at[j]], gt)
            pltpu.sync_copy(gt, accSt.at[dst_v.at[j]], add=True)
            pltpu.sync_copy(ones_v, accDi.at[dst_v.at[j]], add=True)
            pltpu.sync_copy(ones_v, accDo.at[src_v.at[j]], add=True)
            return carry

        lax.fori_loop(0, k, body, 0)
        plsc.subcore_barrier()
        pltpu.sync_copy(accSt.at[pl.ds(r0, rps)], outSt_hbm.at[c, pl.ds(r0, rps)])
        pltpu.sync_copy(accDi.at[pl.ds(r0, rps)], outDi_hbm.at[c, pl.ds(r0, rps)])
        pltpu.sync_copy(accDo.at[pl.ds(r0, rps)], outDo_hbm.at[c, pl.ds(r0, rps)])

    return kern


def _sc_edge(n_pad, d, k):
    rps = n_pad // NS
    ngrp = C // L

    @functools.partial(
        pl.kernel,
        out_type=jax.ShapeDtypeStruct((NC, n_pad, d), F32),
        mesh=_mesh(),
        compiler_params=_SC_PARAMS,
        scratch_types=[
            pltpu.VMEM_SHARED((n_pad, d), F32),
            pltpu.VMEM_SHARED((n_pad, d), F32),
            pltpu.VMEM_SHARED((n_pad, d), F32),
            pltpu.VMEM((k, C), jnp.int32),
            pltpu.VMEM((k, C), jnp.int32),
            pltpu.VMEM((C, d), F32),
            pltpu.VMEM((C, d), F32),
            pltpu.VMEM((C, d), F32),
            pltpu.VMEM((C, d), F32),
            pltpu.VMEM((C, d), F32),
            pltpu.VMEM((C, d), F32),
        ],
    )
    def kern(src_hbm, dst_hbm, h_hbm, m_hbm, zeros_hbm, out_hbm,
             h_s, m_s, acc_s, src_v, dst_v, ha, hb, ma, mb, pos_v, neg_v):
        c = lax.axis_index("c")
        s = lax.axis_index("s")
        wid = c * NS + s
        r0 = s * rps
        pltpu.sync_copy(h_hbm.at[pl.ds(r0, rps)], h_s.at[pl.ds(r0, rps)])
        pltpu.sync_copy(m_hbm.at[pl.ds(r0, rps)], m_s.at[pl.ds(r0, rps)])
        pltpu.sync_copy(zeros_hbm.at[pl.ds(r0, rps)], acc_s.at[pl.ds(r0, rps)])
        pltpu.sync_copy(src_hbm.at[pl.ds(wid * k, k)], src_v)
        pltpu.sync_copy(dst_hbm.at[pl.ds(wid * k, k)], dst_v)
        plsc.subcore_barrier()
        iota = lax.iota(jnp.int32, L)
        shuf = [lax.bitwise_xor(iota, jnp.int32(r)) for r in (1, 2, 4, 8)]

        dnums = lax.GatherDimensionNumbers(
            offset_dims=(), collapsed_slice_dims=(0,), start_index_map=(0,))

        def hsum(v):
            for idx in shuf:
                v = v + lax.gather(
                    v, idx[:, None], dnums, (1,),
                    mode=lax.GatherScatterMode.PROMISE_IN_BOUNDS)
            return v

        def body(j, carry):
            pltpu.sync_copy(h_s.at[src_v.at[j]], ha)
            pltpu.sync_copy(h_s.at[dst_v.at[j]], hb)
            pltpu.sync_copy(m_s.at[src_v.at[j]], ma)
            pltpu.sync_copy(m_s.at[dst_v.at[j]], mb)

            @plsc.parallel_loop(0, C, step=1, unroll=8)
            def _edge(i):
                df = ha[i, :] - hb[i, :]
                d2 = hsum(df * df)
                cc = hsum(ma[i, :] * mb[i, :])
                yi = lax.bitcast_convert_type(
                    jnp.full((L,), 0x5F3759DF, jnp.int32)
                    - lax.shift_right_logical(
                        lax.bitcast_convert_type(d2, jnp.int32), 1),
                    F32)
                hd2 = 0.5 * d2
                yi = yi * (1.5 - hd2 * yi * yi)
                yi = yi * (1.5 - hd2 * yi * yi)
                coef = cc * yi * yi * yi
                v = coef * df
                pos_v[i, :] = v
                neg_v[i, :] = -v

            pltpu.sync_copy(pos_v, acc_s.at[dst_v.at[j]], add=True)
            pltpu.sync_copy(neg_v, acc_s.at[src_v.at[j]], add=True)
            return carry

        lax.fori_loop(0, k, body, 0)
        plsc.subcore_barrier()
        pltpu.sync_copy(acc_s.at[pl.ds(r0, rps)], out_hbm.at[c, pl.ds(r0, rps)])

    return kern


def _tc_call(body, out_shapes, *args):
    return pl.pallas_call(
        body,
        out_shape=tuple(jax.ShapeDtypeStruct(s, F32) for s in out_shapes),
    )(*args)


def _tc_pre(p128, m128):
    def body(p_ref, m_ref, t_ref):
        m = m_ref[...]
        valid = m > 0
        t_ref[...] = jnp.where(valid, p_ref[...] / jnp.where(valid, m, 1.0), 0.0)

    (t,) = _tc_call(body, [p128.shape], p128, m128)
    return t


def _tc1(accSt, accDi, accDo, q128, t128):
    def body(st_ref, di_ref, do_ref, q_ref, t_ref,
             rsin_ref, rsout_ref, x1_ref, dhdp_ref):
        rsin_ref[...] = lax.rsqrt(jnp.maximum(di_ref[0] + di_ref[1], 1.0))
        rsout_ref[...] = lax.rsqrt(jnp.maximum(do_ref[0] + do_ref[1], 1.0))
        x1_ref[...] = q_ref[...] * rsout_ref[...]
        dhdp_ref[...] = st_ref[0] + st_ref[1] + t_ref[...]

    return _tc_call(body, [q128.shape] * 4, accSt, accDi, accDo, q128, t128)


def _tc2(acc, rs_in, rs_out, W1big, b1big, W2big):
    r128, _ = rs_in.shape
    hidb = W1big.shape[1]

    def body(acc_ref, rsin_ref, rsout_ref, w1_ref, b1_ref, w2_ref,
             h2_ref, a_ref):
        xw = (acc_ref[0] + acc_ref[1]) * rsin_ref[...]
        a = jnp.dot(xw, w1_ref[...], preferred_element_type=F32) + b1_ref[...]
        a_ref[...] = a
        h2_ref[...] = jnp.dot(jnp.maximum(a, 0.0), w2_ref[...],
                              preferred_element_type=F32) * rsout_ref[...]

    return _tc_call(body, [rs_in.shape, (r128, hidb)],
                    acc, rs_in, rs_out, W1big, b1big, W2big)


def _tc3(acc, rs_in, q128, b2big):
    def body(acc_ref, rsin_ref, q_ref, b2_ref, h_ref):
        h_ref[...] = ((acc_ref[0] + acc_ref[1]) * rsin_ref[...]
                      + b2_ref[...] + q_ref[...])

    (h,) = _tc_call(body, [q128.shape], acc, rs_in, q128, b2big)
    return h


def _tc5(accG, rs_in, gravity):
    def body(acc_ref, rsin_ref, grav_ref, g_ref, y_ref):
        g = 0.5 * grav_ref[0, 0] * (acc_ref[0] + acc_ref[1])
        g_ref[...] = g
        y_ref[...] = g * rsin_ref[...]

    return _tc_call(body, [rs_in.shape] * 2, accG, rs_in, gravity)


def _tc6(acc, a_big, rs_in, rs_out, W2Tbig, W1Tbig):
    def body(acc_ref, a_ref, rsin_ref, rsout_ref, w2t_ref, w1t_ref, y2_ref):
        zw = (acc_ref[0] + acc_ref[1]) * rsout_ref[...]
        u = jnp.dot(zw, w2t_ref[...], preferred_element_type=F32)
        v = jnp.where(a_ref[...] > 0, u, 0.0)
        y2_ref[...] = jnp.dot(v, w1t_ref[...],
                              preferred_element_type=F32) * rsin_ref[...]

    (y2,) = _tc_call(body, [rs_in.shape], acc, a_big, rs_in, rs_out,
                     W2Tbig, W1Tbig)
    return y2


def _tc7(acc, rs_out, g128):
    def body(acc_ref, rsout_ref, g_ref, dhdq_ref):
        dhdq_ref[...] = (acc_ref[0] + acc_ref[1]) * rsout_ref[...] + g_ref[...]

    (dhdq,) = _tc_call(body, [rs_out.shape], acc, rs_out, g128)
    return dhdq


def kernel(q, p, edge_index, M, W1, b1, W2, b2, gravity):
    n, d = q.shape
    e = edge_index.shape[1]
    nw = NC * NS
    n_pad = -(-(n + 1) // (NS * 8)) * NS * 8
    k = -(-e // (nw * C))
    e_pad = nw * C * k
    r128 = n_pad * d // 128
    nb = 128 // d

    src = edge_index[0].astype(jnp.int32)
    dst = edge_index[1].astype(jnp.int32)
    pad = jnp.full((e_pad - e,), n, jnp.int32)
    srcp = jnp.concatenate([src, pad]).reshape(e_pad // C, C)
    dstp = jnp.concatenate([dst, pad]).reshape(e_pad // C, C)

    m = jnp.diagonal(M, axis1=1, axis2=2)
    m_pad = jnp.pad(m, ((0, n_pad - n), (0, 0)))
    q128 = jnp.pad(q, ((0, n_pad - n), (0, 0))).reshape(r128, 128)
    p128 = jnp.pad(p, ((0, n_pad - n), (0, 0))).reshape(r128, 128)
    m128 = m_pad.reshape(r128, 128)
    zeros16 = jnp.zeros((n_pad, d), F32)

    eye = jnp.eye(nb, dtype=F32)
    W1big = jnp.kron(eye, W1)
    W2big = jnp.kron(eye, W2)
    W2Tbig = jnp.kron(eye, W2.T)
    W1Tbig = jnp.kron(eye, W1.T)
    b1big = jnp.tile(b1, nb).reshape(1, nb * b1.shape[0])
    b2big = jnp.tile(b2, nb).reshape(1, 128)

    def v128(acc):
        return acc.reshape(NC, r128, 128)

    def v16(x):
        return x.reshape(n_pad, d)

    sc_gs = _sc_gs(n_pad, d, k)

    t128 = _tc_pre(p128, m128)
    accSt, accDi, accDo = _sc_prep(n_pad, d, k)(srcp, dstp, v16(t128), zeros16)
    rs_in, rs_out, x1, dhdp = _tc1(v128(accSt), v128(accDi), v128(accDo),
                                   q128, t128)

    acc1 = sc_gs(srcp, dstp, v16(x1), zeros16)
    h2, a_big = _tc2(v128(acc1), rs_in, rs_out, W1big, b1big, W2big)
    acc2 = sc_gs(srcp, dstp, v16(h2), zeros16)
    h = _tc3(v128(acc2), rs_in, q128, b2big)

    accG = _sc_edge(n_pad, d, k)(srcp, dstp, v16(h), m_pad, zeros16)
    g128, y = _tc5(v128(accG), rs_in, gravity)

    accZ = sc_gs(dstp, srcp, v16(y), zeros16)
    y2 = _tc6(v128(accZ), a_big, rs_in, rs_out, W2Tbig, W1Tbig)
    accZ2 = sc_gs(dstp, srcp, v16(y2), zeros16)
    dhdq = _tc7(v128(accZ2), rs_out, g128)

    return jnp.concatenate([v16(dhdq)[:n], v16(dhdp)[:n]], axis=1)
